# Initial kernel scaffold; baseline (speedup 1.0000x reference)
import jax
import jax.numpy as jnp
from jax.experimental import pallas as pl


def kernel(raw, anchors, img_size):
    raise NotImplementedError("write your pallas kernel here")


# speedup vs baseline: 1.3783x; 1.3783x over previous
import functools

import jax
import jax.numpy as jnp
from jax.experimental import pallas as pl
from jax.experimental.pallas import tpu as pltpu


def _decode_block(params_ref, x_ref, o_ref, *, nG: int, nCH: int):
    a = pl.program_id(1)
    n = nG * nG
    x = x_ref[0, 0]
    aw = params_ref[2 * a]
    ah = params_ref[2 * a + 1]
    stride = params_ref[6]
    clipmax = params_ref[7]

    g = jax.lax.broadcasted_iota(jnp.int32, (1, n), 1)
    gx = (g % nG).astype(jnp.float32)
    gy = (g // nG).astype(jnp.float32)
    cx = (gx + 0.5) * stride
    cy = (gy + 0.5) * stride

    e = jnp.exp(x[0:4, :])
    l = jnp.clip(e[0:1] * aw, 0.0, clipmax)
    t = jnp.clip(e[1:2] * ah, 0.0, clipmax)
    r = jnp.clip(e[2:3] * aw, 0.0, clipmax)
    b = jnp.clip(e[3:4] * ah, 0.0, clipmax)
    xc = cx + (r - l) * 0.5
    yc = cy + (b - t) * 0.5
    w = l + r
    h = t + b

    sig = jax.nn.sigmoid(x[4:, :])
    y = jnp.concatenate([xc, yc, w, h, sig], axis=0)
    o_ref[0, 0] = y.T


def kernel(raw, anchors, img_size):
    nB, C, nG, _ = raw.shape
    nA = anchors.shape[0]
    nCH = C // nA
    n = nG * nG
    img = jnp.asarray(img_size)
    stride = (img // nG).astype(jnp.float32)
    clipmax = img.astype(jnp.float32)
    params = jnp.concatenate(
        [anchors.reshape(-1).astype(jnp.float32), jnp.stack([stride, clipmax])]
    )
    x = raw.reshape(nB, nA, nCH, n)
    out = pl.pallas_call(
        functools.partial(_decode_block, nG=nG, nCH=nCH),
        grid=(nB, nA),
        in_specs=[
            pl.BlockSpec(memory_space=pltpu.SMEM),
            pl.BlockSpec((1, 1, nCH, n), lambda b, a: (b, a, 0, 0)),
        ],
        out_specs=pl.BlockSpec((1, 1, n, nCH), lambda b, a: (b, a, 0, 0)),
        out_shape=jax.ShapeDtypeStruct((nB, nA, n, nCH), jnp.float32),
    )(params, x)
    return out.reshape(nB, nA * n, nCH)


# speedup vs baseline: 1.9740x; 1.4321x over previous
import functools

import jax
import jax.numpy as jnp
from jax.experimental import pallas as pl
from jax.experimental.pallas import tpu as pltpu


def _decode_block(params_ref, x_ref, o_ref, *, nG: int, nCH: int):
    a = pl.program_id(1)
    n = nG * nG
    x = x_ref[0].reshape(nCH, n)
    aw = params_ref[2 * a]
    ah = params_ref[2 * a + 1]
    stride = params_ref[6]
    clipmax = params_ref[7]

    g = jax.lax.broadcasted_iota(jnp.int32, (1, n), 1)
    gx = (g % nG).astype(jnp.float32)
    gy = (g // nG).astype(jnp.float32)
    cx = (gx + 0.5) * stride
    cy = (gy + 0.5) * stride

    e = jnp.exp(x[0:4, :])
    l = jnp.clip(e[0:1] * aw, 0.0, clipmax)
    t = jnp.clip(e[1:2] * ah, 0.0, clipmax)
    r = jnp.clip(e[2:3] * aw, 0.0, clipmax)
    b = jnp.clip(e[3:4] * ah, 0.0, clipmax)
    xc = cx + (r - l) * 0.5
    yc = cy + (b - t) * 0.5
    w = l + r
    h = t + b

    sig = jax.nn.sigmoid(x[4:, :])
    y = jnp.concatenate([xc, yc, w, h, sig], axis=0)
    o_ref[0] = y.T


def kernel(raw, anchors, img_size):
    nB, C, nG, _ = raw.shape
    nA = anchors.shape[0]
    nCH = C // nA
    n = nG * nG
    img = jnp.asarray(img_size)
    stride = (img // nG).astype(jnp.float32)
    clipmax = img.astype(jnp.float32)
    params = jnp.concatenate(
        [anchors.reshape(-1).astype(jnp.float32), jnp.stack([stride, clipmax])]
    )
    out = pl.pallas_call(
        functools.partial(_decode_block, nG=nG, nCH=nCH),
        grid=(nB, nA),
        in_specs=[
            pl.BlockSpec(memory_space=pltpu.SMEM),
            pl.BlockSpec((1, nCH, nG, nG), lambda b, a: (b, a, 0, 0)),
        ],
        out_specs=pl.BlockSpec((1, n, nCH), lambda b, a: (b, a, 0)),
        out_shape=jax.ShapeDtypeStruct((nB, nA * n, nCH), jnp.float32),
    )(params, raw)
    return out


# speedup vs baseline: 7.6671x; 3.8841x over previous
import functools

import jax
import jax.numpy as jnp
from jax.experimental import pallas as pl
from jax.experimental.pallas import tpu as pltpu


def _decode_block(params_ref, x_ref, o_ref, y_ref, sem, *, nG, nCH, nA, rows):
    b = pl.program_id(0)
    j = pl.program_id(1)
    n = rows * nG
    stride = params_ref[2 * nA]
    clipmax = params_ref[2 * nA + 1]

    x = x_ref[0].reshape(n, nA * nCH)
    xt = x.T

    g = jax.lax.broadcasted_iota(jnp.int32, (1, n), 1)
    gx = (g % nG).astype(jnp.float32)
    gy = (j * rows + g // nG).astype(jnp.float32)
    cx = (gx + 0.5) * stride
    cy = (gy + 0.5) * stride

    for a in range(nA):
        base = a * nCH
        aw = params_ref[2 * a]
        ah = params_ref[2 * a + 1]
        e = jnp.exp(xt[base : base + 4, :])
        l = jnp.clip(e[0:1] * aw, 0.0, clipmax)
        t = jnp.clip(e[1:2] * ah, 0.0, clipmax)
        r = jnp.clip(e[2:3] * aw, 0.0, clipmax)
        bb = jnp.clip(e[3:4] * ah, 0.0, clipmax)
        xc = cx + (r - l) * 0.5
        yc = cy + (bb - t) * 0.5
        w = l + r
        h = t + bb
        sig = jax.nn.sigmoid(xt[base + 4 : base + nCH, :])
        y_ref[a] = jnp.concatenate([xc, yc, w, h, sig], axis=0)

    for a in range(nA):
        cp = pltpu.make_async_copy(
            y_ref.at[a],
            o_ref.at[:, b, pl.ds(a * nG * nG + j * n, n)],
            sem,
        )
        cp.start()
    for a in range(nA):
        pltpu.make_async_copy(
            y_ref.at[a],
            o_ref.at[:, b, pl.ds(a * nG * nG + j * n, n)],
            sem,
        ).wait()


def kernel(raw, anchors, img_size):
    nB, C, nG, _ = raw.shape
    nA = anchors.shape[0]
    nCH = C // nA
    img = jnp.asarray(img_size)
    stride = (img // nG).astype(jnp.float32)
    clipmax = img.astype(jnp.float32)
    params = jnp.concatenate(
        [anchors.reshape(-1).astype(jnp.float32), jnp.stack([stride, clipmax])]
    )
    x = jnp.transpose(raw, (0, 2, 3, 1))
    rows = 32
    out = pl.pallas_call(
        functools.partial(_decode_block, nG=nG, nCH=nCH, nA=nA, rows=rows),
        grid=(nB, nG // rows),
        in_specs=[
            pl.BlockSpec(memory_space=pltpu.SMEM),
            pl.BlockSpec((1, rows, nG, C), lambda b, j: (b, j, 0, 0)),
        ],
        out_specs=pl.BlockSpec(memory_space=pl.ANY),
        out_shape=jax.ShapeDtypeStruct((nCH, nB, nA * nG * nG), jnp.float32),
        scratch_shapes=[
            pltpu.VMEM((nA, nCH, rows * nG), jnp.float32),
            pltpu.SemaphoreType.DMA,
        ],
    )(params, x)
    return jnp.transpose(out, (1, 2, 0))


# speedup vs baseline: 10.1253x; 1.3206x over previous
import functools

import jax
import jax.numpy as jnp
from jax.experimental import pallas as pl
from jax.experimental.pallas import tpu as pltpu


def _decode_block(
    params_ref, x_ref, o_ref, y_ref, sems, *, nG, nCH, nA, rows, nprog
):
    b = pl.program_id(0)
    j = pl.program_id(1)
    nj = pl.num_programs(1)
    pid = b * nj + j
    par = jax.lax.rem(pid, 2)
    n = rows * nG
    stride = params_ref[2 * nA]
    clipmax = params_ref[2 * nA + 1]

    def _dma(src_pid, buf, a):
        bb2 = src_pid // nj
        jj2 = jax.lax.rem(src_pid, nj)
        return pltpu.make_async_copy(
            y_ref.at[buf, a],
            o_ref.at[:, bb2, pl.ds(a * nG * nG + jj2 * n, n)],
            sems.at[buf],
        )

    @pl.when(pid >= 2)
    def _():
        for a in range(nA):
            _dma(pid - 2, par, a).wait()

    x = x_ref[0].reshape(n, nA * nCH)
    xt = x.T

    g = jax.lax.broadcasted_iota(jnp.int32, (1, n), 1)
    gx = (g % nG).astype(jnp.float32)
    gy = (j * rows + g // nG).astype(jnp.float32)
    cx = (gx + 0.5) * stride
    cy = (gy + 0.5) * stride

    for a in range(nA):
        base = a * nCH
        aw = params_ref[2 * a]
        ah = params_ref[2 * a + 1]
        e = jnp.exp(xt[base : base + 4, :])
        l = jnp.clip(e[0:1] * aw, 0.0, clipmax)
        t = jnp.clip(e[1:2] * ah, 0.0, clipmax)
        r = jnp.clip(e[2:3] * aw, 0.0, clipmax)
        bb = jnp.clip(e[3:4] * ah, 0.0, clipmax)
        xc = cx + (r - l) * 0.5
        yc = cy + (bb - t) * 0.5
        w = l + r
        h = t + bb
        sig = jax.nn.sigmoid(xt[base + 4 : base + nCH, :])
        y_ref[par, a] = jnp.concatenate([xc, yc, w, h, sig], axis=0)

    for a in range(nA):
        _dma(pid, par, a).start()

    @pl.when(pid == nprog - 1)
    def _():
        if nprog >= 2:
            for a in range(nA):
                _dma(pid - 1, 1 - par, a).wait()
        for a in range(nA):
            _dma(pid, par, a).wait()


def kernel(raw, anchors, img_size):
    nB, C, nG, _ = raw.shape
    nA = anchors.shape[0]
    nCH = C // nA
    img = jnp.asarray(img_size)
    stride = (img // nG).astype(jnp.float32)
    clipmax = img.astype(jnp.float32)
    params = jnp.concatenate(
        [anchors.reshape(-1).astype(jnp.float32), jnp.stack([stride, clipmax])]
    )
    x = jnp.transpose(raw, (0, 2, 3, 1))
    rows = 32
    nprog = nB * (nG // rows)
    out = pl.pallas_call(
        functools.partial(
            _decode_block, nG=nG, nCH=nCH, nA=nA, rows=rows, nprog=nprog
        ),
        grid=(nB, nG // rows),
        in_specs=[
            pl.BlockSpec(memory_space=pltpu.SMEM),
            pl.BlockSpec((1, rows, nG, C), lambda b, j: (b, j, 0, 0)),
        ],
        out_specs=pl.BlockSpec(memory_space=pl.ANY),
        out_shape=jax.ShapeDtypeStruct((nCH, nB, nA * nG * nG), jnp.float32),
        scratch_shapes=[
            pltpu.VMEM((2, nA, nCH, rows * nG), jnp.float32),
            pltpu.SemaphoreType.DMA((2,)),
        ],
    )(params, x)
    return jnp.transpose(out, (1, 2, 0))


# speedup vs baseline: 11.7250x; 1.1580x over previous
import functools

import jax
import jax.numpy as jnp
from jax.experimental import pallas as pl
from jax.experimental.pallas import tpu as pltpu


def _decode_block(
    params_ref, x_ref, o_ref, y_ref, sems, *, nG, nCH, nA, rows, nprog
):
    b = pl.program_id(0)
    j = pl.program_id(1)
    nj = pl.num_programs(1)
    pid = b * nj + j
    par = jax.lax.rem(pid, 2)
    n = rows * nG
    stride = params_ref[2 * nA]
    clipmax = params_ref[2 * nA + 1]

    def _dma(src_pid, buf, a):
        bb2 = src_pid // nj
        jj2 = jax.lax.rem(src_pid, nj)
        return pltpu.make_async_copy(
            y_ref.at[buf, a],
            o_ref.at[:, bb2, pl.ds(a * nG * nG + jj2 * n, n)],
            sems.at[buf],
        )

    @pl.when(pid >= 2)
    def _():
        for a in range(nA):
            _dma(pid - 2, par, a).wait()

    x = x_ref[0].reshape(n, nA * nCH)
    xt = x.T

    g = jax.lax.broadcasted_iota(jnp.int32, (1, n), 1)
    gx = (g % nG).astype(jnp.float32)
    gy = (j * rows + g // nG).astype(jnp.float32)
    cx = (gx + 0.5) * stride
    cy = (gy + 0.5) * stride

    for a in range(nA):
        base = a * nCH
        aw = params_ref[2 * a]
        ah = params_ref[2 * a + 1]
        e = jnp.exp(xt[base : base + 4, :])
        l = jnp.clip(e[0:1] * aw, 0.0, clipmax)
        t = jnp.clip(e[1:2] * ah, 0.0, clipmax)
        r = jnp.clip(e[2:3] * aw, 0.0, clipmax)
        bb = jnp.clip(e[3:4] * ah, 0.0, clipmax)
        xc = cx + (r - l) * 0.5
        yc = cy + (bb - t) * 0.5
        w = l + r
        h = t + bb
        sig = jax.nn.sigmoid(xt[base + 4 : base + nCH, :])
        y_ref[par, a] = jnp.concatenate([xc, yc, w, h, sig], axis=0)

    for a in range(nA):
        _dma(pid, par, a).start()

    @pl.when(pid == nprog - 1)
    def _():
        if nprog >= 2:
            for a in range(nA):
                _dma(pid - 1, 1 - par, a).wait()
        for a in range(nA):
            _dma(pid, par, a).wait()


def kernel(raw, anchors, img_size):
    nB, C, nG, _ = raw.shape
    nA = anchors.shape[0]
    nCH = C // nA
    img = jnp.asarray(img_size)
    stride = (img // nG).astype(jnp.float32)
    clipmax = img.astype(jnp.float32)
    params = jnp.concatenate(
        [anchors.reshape(-1).astype(jnp.float32), jnp.stack([stride, clipmax])]
    )
    x = jnp.transpose(raw, (0, 2, 3, 1))
    rows = 64
    nprog = nB * (nG // rows)
    out = pl.pallas_call(
        functools.partial(
            _decode_block, nG=nG, nCH=nCH, nA=nA, rows=rows, nprog=nprog
        ),
        grid=(nB, nG // rows),
        in_specs=[
            pl.BlockSpec(memory_space=pltpu.SMEM),
            pl.BlockSpec((1, rows, nG, C), lambda b, j: (b, j, 0, 0)),
        ],
        out_specs=pl.BlockSpec(memory_space=pl.ANY),
        out_shape=jax.ShapeDtypeStruct((nCH, nB, nA * nG * nG), jnp.float32),
        scratch_shapes=[
            pltpu.VMEM((2, nA, nCH, rows * nG), jnp.float32),
            pltpu.SemaphoreType.DMA((2,)),
        ],
    )(params, x)
    return jnp.transpose(out, (1, 2, 0))


# speedup vs baseline: 11.7893x; 1.0055x over previous
import functools

import jax
import jax.numpy as jnp
from jax.experimental import pallas as pl
from jax.experimental.pallas import tpu as pltpu


def _decode_block(params_ref, x_ref, o_ref, y_ref, sems, *, nG, nCH, nA, nB):
    b = pl.program_id(0)
    par = jax.lax.rem(b, 2)
    n = nG * nG
    stride = params_ref[2 * nA]
    clipmax = params_ref[2 * nA + 1]

    def _dma(src_b, buf):
        return pltpu.make_async_copy(
            y_ref.at[buf], o_ref.at[:, src_b, :], sems.at[buf]
        )

    @pl.when(b >= 2)
    def _():
        _dma(b - 2, par).wait()

    x = x_ref[0].reshape(n, nA * nCH)
    xt = x.T

    g = jax.lax.broadcasted_iota(jnp.int32, (1, n), 1)
    gx = (g % nG).astype(jnp.float32)
    gy = (g // nG).astype(jnp.float32)
    cx = (gx + 0.5) * stride
    cy = (gy + 0.5) * stride

    for a in range(nA):
        base = a * nCH
        aw = params_ref[2 * a]
        ah = params_ref[2 * a + 1]
        e = jnp.exp(xt[base : base + 4, :])
        l = jnp.clip(e[0:1] * aw, 0.0, clipmax)
        t = jnp.clip(e[1:2] * ah, 0.0, clipmax)
        r = jnp.clip(e[2:3] * aw, 0.0, clipmax)
        bb = jnp.clip(e[3:4] * ah, 0.0, clipmax)
        xc = cx + (r - l) * 0.5
        yc = cy + (bb - t) * 0.5
        w = l + r
        h = t + bb
        sig = jax.nn.sigmoid(xt[base + 4 : base + nCH, :])
        y_ref[par, :, a * n : (a + 1) * n] = jnp.concatenate(
            [xc, yc, w, h, sig], axis=0
        )

    _dma(b, par).start()

    @pl.when(b == nB - 1)
    def _():
        if nB >= 2:
            _dma(b - 1, 1 - par).wait()
        _dma(b, par).wait()


def kernel(raw, anchors, img_size):
    nB, C, nG, _ = raw.shape
    nA = anchors.shape[0]
    nCH = C // nA
    img = jnp.asarray(img_size)
    stride = (img // nG).astype(jnp.float32)
    clipmax = img.astype(jnp.float32)
    params = jnp.concatenate(
        [anchors.reshape(-1).astype(jnp.float32), jnp.stack([stride, clipmax])]
    )
    x = jnp.transpose(raw, (0, 2, 3, 1))
    out = pl.pallas_call(
        functools.partial(_decode_block, nG=nG, nCH=nCH, nA=nA, nB=nB),
        grid=(nB,),
        in_specs=[
            pl.BlockSpec(memory_space=pltpu.SMEM),
            pl.BlockSpec((1, nG, nG, C), lambda b: (b, 0, 0, 0)),
        ],
        out_specs=pl.BlockSpec(memory_space=pl.ANY),
        out_shape=jax.ShapeDtypeStruct((nCH, nB, nA * nG * nG), jnp.float32),
        scratch_shapes=[
            pltpu.VMEM((2, nCH, nA * nG * nG), jnp.float32),
            pltpu.SemaphoreType.DMA((2,)),
        ],
    )(params, x)
    return jnp.transpose(out, (1, 2, 0))
